# baseline (speedup 1.0000x reference)
import functools

import numpy as np

import jax
import jax.numpy as jnp
from jax import lax
from jax.experimental import pallas as pl
from jax.experimental.pallas import tpu as pltpu
from jax.experimental.pallas import tpu_sc as plsc

N = 50000
E = 800000
VOCAB = 1000
EMB = 32
HID = 64
NCLS = 10
G = 256

NCORE = 2
NSUB = 16

NROW = 392
NROWB = 49
N_PAD = NROW * 128
DEAD = N_PAD - N
EROW = 6400
E_PAD = EROW * 128
TROW = EROW // NSUB
GRP = 8
NGRP = TROW // GRP
DROW = EROW // (NCORE * NSUB)
NODES_T = N_PAD // NSUB

_mesh = plsc.VectorSubcoreMesh(core_axis_name="c", subcore_axis_name="s",
                               num_cores=NCORE, num_subcores=NSUB)

_HIGH = lax.Precision.HIGHEST

_SC_PARAMS = pltpu.CompilerParams(use_tc_tiling_on_sc=False)


@functools.partial(
    pl.kernel,
    out_type=(jax.ShapeDtypeStruct((N_PAD, EMB), jnp.float32),
              jax.ShapeDtypeStruct((N_PAD,), jnp.float32),
              jax.ShapeDtypeStruct((N_PAD,), jnp.float32)),
    mesh=_mesh,
    compiler_params=_SC_PARAMS,
    scratch_types=[
        pltpu.VMEM((8, 128), jnp.int32),
        pltpu.VMEM((8, 128, EMB), jnp.float32),
        pltpu.VMEM((128,), jnp.float32),
        pltpu.VMEM((DROW, 128), jnp.int32),
        pltpu.VMEM((NODES_T,), jnp.float32),
        pltpu.VMEM_SHARED((N_PAD,), jnp.float32),
        pltpu.SemaphoreType.DMA,
        pltpu.SemaphoreType.DMA,
    ],
)
def _emb_deg(tok_hbm, dst_hbm, emb_hbm, x_out, deg0_out, deg1_out,
             tokb_v, erows_v, ones_v, dstb_v, zeros_v, deg_sh, sem, dsem):
    c = lax.axis_index("c")
    s = lax.axis_index("s")
    w = s * NCORE + c

    for g in range(2):
        q = g * 32 + w

        @pl.when(q < NROWB)
        def _():
            pltpu.sync_copy(tok_hbm.at[q], tokb_v)
            cps = [pltpu.async_copy(emb_hbm.at[tokb_v.at[j]], erows_v.at[j],
                                    sem) for j in range(8)]
            for cp in cps:
                cp.wait()
            for j in range(8):
                pltpu.sync_copy(erows_v.at[j],
                                x_out.at[pl.ds(q * 1024 + j * 128, 128)])

    def _z(i, _):
        zeros_v[pl.ds(i * 16, 16)] = jnp.zeros((16,), jnp.float32)
        return 0
    lax.fori_loop(0, NODES_T // 16, _z, 0)

    def _o(i, _):
        ones_v[pl.ds(i * 16, 16)] = jnp.ones((16,), jnp.float32)
        return 0
    lax.fori_loop(0, 8, _o, 0)

    pltpu.sync_copy(zeros_v, deg_sh.at[pl.ds(s * NODES_T, NODES_T)])
    plsc.subcore_barrier()

    base = w * DROW
    pltpu.sync_copy(dst_hbm.at[pl.ds(base, DROW)], dstb_v)

    def _sc(j, _):
        pltpu.async_copy(ones_v, deg_sh.at[dstb_v.at[j]], dsem, add=True)
        return 0
    lax.fori_loop(0, DROW, _sc, 0)

    def _dr(j, _):
        pltpu.make_async_copy(ones_v, deg_sh.at[dstb_v.at[j]], dsem).wait()
        return 0
    lax.fori_loop(0, DROW, _dr, 0)
    plsc.subcore_barrier()

    @pl.when(c == 0)
    def _():
        pltpu.sync_copy(deg_sh.at[pl.ds(s * NODES_T, NODES_T)],
                        deg0_out.at[pl.ds(s * NODES_T, NODES_T)])

    @pl.when(c == 1)
    def _():
        pltpu.sync_copy(deg_sh.at[pl.ds(s * NODES_T, NODES_T)],
                        deg1_out.at[pl.ds(s * NODES_T, NODES_T)])


F2 = 16


def _make_conv(f2, npass):
    n_io = NCORE * npass
    NS = 8
    CH = [(0, 136), (136, 136), (272, 128)]
    CBUF = 136

    @functools.partial(
        pl.kernel,
        out_type=tuple(jax.ShapeDtypeStruct((N_PAD, f2), jnp.float32)
                       for _ in range(n_io)),
        mesh=_mesh,
        compiler_params=_SC_PARAMS,
        scratch_types=[
            pltpu.VMEM((CBUF, 128), jnp.int32),
            pltpu.VMEM((CBUF, 128), jnp.int32),
            pltpu.VMEM((NS, 128, f2), jnp.float32),
            pltpu.VMEM_SHARED((N_PAD, f2), jnp.float32),
        ] + [pltpu.SemaphoreType.DMA] * (2 * NS),
    )
    def conv(*args):
        src_hbm, dst_hbm = args[0], args[1]
        g_hbms = args[2:2 + n_io]
        outs = args[2 + n_io:2 + 2 * n_io]
        rest = args[2 + 2 * n_io:]
        idxs_v, idxd_v, rows_v, acc_sh = rest[:4]
        sem_g = rest[4:4 + NS]
        sem_s = rest[4 + NS:]
        c = lax.axis_index("c")
        s = lax.axis_index("s")
        nb = s * NODES_T
        eb = s * TROW

        def run(g_hbm, out_hbm):
            pltpu.sync_copy(g_hbm.at[pl.ds(nb, NODES_T)],
                            acc_sh.at[pl.ds(nb, NODES_T)])
            plsc.subcore_barrier()

            def fire_g(r, u):
                pltpu.async_copy(g_hbm.at[idxs_v.at[r]], rows_v.at[u],
                                 sem_g[u])

            def drain_g(u):
                pltpu.make_async_copy(g_hbm.at[idxs_v.at[0]], rows_v.at[u],
                                      sem_g[u]).wait()

            def fire_s(r, u):
                pltpu.async_copy(rows_v.at[u], acc_sh.at[idxd_v.at[r]],
                                 sem_s[u], add=True)

            def drain_s(u):
                pltpu.make_async_copy(rows_v.at[u], acc_sh.at[idxd_v.at[0]],
                                      sem_s[u]).wait()

            for coff, clen in CH:
                pltpu.sync_copy(src_hbm.at[pl.ds(eb + coff, clen)],
                                idxs_v.at[pl.ds(0, clen)])
                pltpu.sync_copy(dst_hbm.at[pl.ds(eb + coff, clen)],
                                idxd_v.at[pl.ds(0, clen)])

                def octet(i, _):
                    for u in range(NS):
                        r = NS * i + u
                        v = (u + NS - 1) % NS

                        @pl.when(r >= NS)
                        def _():
                            drain_s(u)
                        fire_g(r, u)

                        @pl.when(r >= 1)
                        def _():
                            drain_g(v)
                            fire_s(r - 1, v)
                    return 0

                lax.fori_loop(0, clen // NS, octet, 0)
                last = (clen - 1) % NS
                drain_g(last)
                fire_s(clen - 1, last)
                for b in range(NS):
                    drain_s(b)

            plsc.subcore_barrier()
            pltpu.sync_copy(acc_sh.at[pl.ds(nb, NODES_T)],
                            out_hbm.at[pl.ds(nb, NODES_T)])

        for ci in range(NCORE):
            @pl.when(c == ci)
            def _():
                for p in range(npass):
                    run(g_hbms[ci * npass + p], outs[ci * npass + p])

    return conv


_conv1 = _make_conv(16, 1)
_conv2 = _make_conv(16, 2)


_BLK = 512
_NBLK = N_PAD // _BLK
_XB = _BLK * EMB // 128
_QB = _BLK * F2 // 128

_P4 = (np.arange(_BLK)[None, :] // 4 == np.arange(128)[:, None]
       ).astype(np.float32)
_K4 = (np.arange(128)[None, :] // 32 == (np.arange(_BLK) % 4)[:, None]
       ).astype(np.float32)
_P8 = (np.arange(_BLK)[None, :] // 8 == np.arange(64)[:, None]
       ).astype(np.float32)
_K8 = (np.arange(512)[None, :] // 64 == (np.arange(_BLK) % 8)[:, None]
       ).astype(np.float32)
_E2 = [(np.arange(128)[None, :] == 2 * np.arange(64)[:, None] + t
        ).astype(np.float32) for t in range(2)]


def _mk_pi(off, t):
    lo = np.arange(128)[None, :]
    li = np.arange(128)[:, None]
    j, f = lo // 16, lo % 16
    return ((j // 4 == t) & (li == (j % 4) * 32 + off + f)).astype(np.float32)


_PI = {(off, t): _mk_pi(off, t) for off in (0, F2) for t in range(2)}


def _mk_piq(k):
    lo = np.arange(128)[None, :]
    li = np.arange(512)[:, None]
    return (li == (lo // 16) * 64 + k * 16 + lo % 16).astype(np.float32)


_PIQ = [_mk_piq(k) for k in range(4)]


def _dot(a, b):
    return jnp.dot(a, b, precision=_HIGH, preferred_element_type=jnp.float32)


def _expand(dinv, pmat, kmat):
    return _dot(pmat * dinv[None, :], kmat)


def _prep_body(deg0_ref, deg1_ref, xp_ref, p4_ref, k4_ref, e0_ref, e1_ref,
               pa0_ref, pa1_ref, pb0_ref, pb1_ref,
               dinv_ref, g1a_ref, g1b_ref):
    deg = deg0_ref[...] + deg1_ref[...] + 1.0
    dinv = lax.rsqrt(deg)
    dinv_ref[...] = dinv
    gp = xp_ref[...] * _expand(dinv, p4_ref[...], k4_ref[...])
    e = (e0_ref[...], e1_ref[...])
    g1a_ref[...] = (_dot(_dot(e[0], gp), pa0_ref[...])
                    + _dot(_dot(e[1], gp), pa1_ref[...]))
    g1b_ref[...] = (_dot(_dot(e[0], gp), pb0_ref[...])
                    + _dot(_dot(e[1], gp), pb1_ref[...]))


_prep = pl.pallas_call(
    _prep_body,
    grid=(_NBLK,),
    in_specs=[
        pl.BlockSpec((_BLK,), lambda i: (i,)),
        pl.BlockSpec((_BLK,), lambda i: (i,)),
        pl.BlockSpec((_XB, 128), lambda i: (i, 0)),
        pl.BlockSpec((128, _BLK), lambda i: (0, 0)),
        pl.BlockSpec((_BLK, 128), lambda i: (0, 0)),
        pl.BlockSpec((_QB, 128), lambda i: (0, 0)),
        pl.BlockSpec((_QB, 128), lambda i: (0, 0)),
        pl.BlockSpec((128, 128), lambda i: (0, 0)),
        pl.BlockSpec((128, 128), lambda i: (0, 0)),
        pl.BlockSpec((128, 128), lambda i: (0, 0)),
        pl.BlockSpec((128, 128), lambda i: (0, 0)),
    ],
    out_specs=(
        pl.BlockSpec((_BLK,), lambda i: (i,)),
        pl.BlockSpec((_QB, 128), lambda i: (i, 0)),
        pl.BlockSpec((_QB, 128), lambda i: (i, 0)),
    ),
    out_shape=(jax.ShapeDtypeStruct((N_PAD,), jnp.float32),
               jax.ShapeDtypeStruct((N_PAD * F2 // 128, 128), jnp.float32),
               jax.ShapeDtypeStruct((N_PAD * F2 // 128, 128), jnp.float32)))


def _mid_body(s1a_ref, s1b_ref, dinv_ref, bd1a_ref, bd1b_ref, b1t_ref,
              p8_ref, k8_ref, piq0_ref, piq1_ref, piq2_ref, piq3_ref,
              q0_ref, q1_ref, q2_ref, q3_ref):
    dinv = dinv_ref[...]
    y = _dot(s1a_ref[...], bd1a_ref[...]) + _dot(s1b_ref[...], bd1b_ref[...])
    dinv64 = _expand(dinv, p8_ref[...], k8_ref[...])
    x1 = jnp.maximum(y * dinv64 + b1t_ref[...][None, :], 0.0)
    g2 = x1 * dinv64
    q0_ref[...] = _dot(g2, piq0_ref[...])
    q1_ref[...] = _dot(g2, piq1_ref[...])
    q2_ref[...] = _dot(g2, piq2_ref[...])
    q3_ref[...] = _dot(g2, piq3_ref[...])


_mid = pl.pallas_call(
    _mid_body,
    grid=(_NBLK,),
    in_specs=[
        pl.BlockSpec((_QB, 128), lambda i: (i, 0)),
        pl.BlockSpec((_QB, 128), lambda i: (i, 0)),
        pl.BlockSpec((_BLK,), lambda i: (i,)),
        pl.BlockSpec((128, 512), lambda i: (0, 0)),
        pl.BlockSpec((128, 512), lambda i: (0, 0)),
        pl.BlockSpec((512,), lambda i: (0,)),
        pl.BlockSpec((_QB, _BLK), lambda i: (0, 0)),
        pl.BlockSpec((_BLK, 512), lambda i: (0, 0)),
        pl.BlockSpec((512, 128), lambda i: (0, 0)),
        pl.BlockSpec((512, 128), lambda i: (0, 0)),
        pl.BlockSpec((512, 128), lambda i: (0, 0)),
        pl.BlockSpec((512, 128), lambda i: (0, 0)),
    ],
    out_specs=tuple(pl.BlockSpec((_QB, 128), lambda i: (i, 0))
                    for _ in range(4)),
    out_shape=tuple(jax.ShapeDtypeStruct((N_PAD * F2 // 128, 128),
                                         jnp.float32)
                    for _ in range(4)))


def _fin_body(q0_ref, q1_ref, q2_ref, q3_ref, dinv_ref, bat2_ref,
              bd0_ref, bd1_ref, bd2_ref, bd3_ref, b2t_ref,
              p8_ref, k8_ref, wl_ref, bl_ref, out_ref, acc_s, acc_c):
    i = pl.program_id(0)

    @pl.when(i == 0)
    def _():
        acc_s[...] = jnp.zeros_like(acc_s)
        acc_c[...] = jnp.zeros_like(acc_c)

    y = (_dot(q0_ref[...], bd0_ref[...]) + _dot(q1_ref[...], bd1_ref[...])
         + _dot(q2_ref[...], bd2_ref[...]) + _dot(q3_ref[...], bd3_ref[...]))
    dinv64 = _expand(dinv_ref[...], p8_ref[...], k8_ref[...])
    x2 = jnp.maximum(y * dinv64 + b2t_ref[...][None, :], 0.0)
    bat2 = bat2_ref[...]
    for j in range(8):
        bj = bat2[:, j]
        oh = (bj[:, None]
              == lax.broadcasted_iota(jnp.int32, (_QB, G), 1)
              ).astype(jnp.float32)
        acc_s[...] += lax.dot_general(
            oh, x2[:, j * HID:(j + 1) * HID], (((0,), (0,)), ((), ())),
            precision=_HIGH, preferred_element_type=jnp.float32)
        acc_c[...] += jnp.sum(oh, axis=0)

    @pl.when(i == pl.num_programs(0) - 1)
    def _():
        pooled = acc_s[...] / jnp.maximum(acc_c[...], 1.0)[:, None]
        out_ref[...] = (jnp.dot(pooled, wl_ref[...], precision=_HIGH,
                                preferred_element_type=jnp.float32)
                        + bl_ref[...][None, :])


_fin = pl.pallas_call(
    _fin_body,
    grid=(_NBLK,),
    in_specs=[
        pl.BlockSpec((_QB, 128), lambda i: (i, 0)),
        pl.BlockSpec((_QB, 128), lambda i: (i, 0)),
        pl.BlockSpec((_QB, 128), lambda i: (i, 0)),
        pl.BlockSpec((_QB, 128), lambda i: (i, 0)),
        pl.BlockSpec((_BLK,), lambda i: (i,)),
        pl.BlockSpec((_QB, 8), lambda i: (i, 0)),
        pl.BlockSpec((128, 512), lambda i: (0, 0)),
        pl.BlockSpec((128, 512), lambda i: (0, 0)),
        pl.BlockSpec((128, 512), lambda i: (0, 0)),
        pl.BlockSpec((128, 512), lambda i: (0, 0)),
        pl.BlockSpec((512,), lambda i: (0,)),
        pl.BlockSpec((_QB, _BLK), lambda i: (0, 0)),
        pl.BlockSpec((_BLK, 512), lambda i: (0, 0)),
        pl.BlockSpec((HID, NCLS), lambda i: (0, 0)),
        pl.BlockSpec((NCLS,), lambda i: (0,)),
    ],
    out_specs=pl.BlockSpec((G, NCLS), lambda i: (0, 0)),
    out_shape=jax.ShapeDtypeStruct((G, NCLS), jnp.float32),
    scratch_shapes=[pltpu.VMEM((G, HID), jnp.float32),
                    pltpu.VMEM((G,), jnp.float32)],
)


def kernel(x_token, edge_index, batch, embed, W1, b1, W2, b2, Wl, bl):
    npad = E_PAD - E
    dead = N + (jnp.arange(npad, dtype=jnp.int32) % DEAD)
    src = jnp.concatenate([edge_index[0], dead]).reshape(EROW, 128)
    dst = jnp.concatenate([edge_index[1], dead]).reshape(EROW, 128)
    tok = jnp.concatenate(
        [x_token, jnp.zeros((DEAD,), jnp.int32)]).reshape(NROWB, 8, 128)
    bat2 = jnp.concatenate(
        [batch, jnp.full((DEAD,), G, jnp.int32)]).reshape(N_PAD // 8, 8)

    def up(a):
        return jnp.reshape(a, (N_PAD, F2))

    def dn(a):
        return jnp.reshape(a, (N_PAD * F2 // 128, 128))

    eye8 = jnp.eye(8, dtype=jnp.float32)
    bd1a = jnp.kron(eye8, W1[:F2])
    bd1b = jnp.kron(eye8, W1[F2:])
    bd2 = [jnp.kron(eye8, W2[k * F2:(k + 1) * F2]) for k in range(4)]
    b1t = jnp.tile(b1, 8)
    b2t = jnp.tile(b2, 8)

    x, deg0, deg1 = _emb_deg(tok, dst, embed)
    xp = jnp.reshape(x, (N_PAD * EMB // 128, 128))
    dinv, g1a, g1b = _prep(deg0, deg1, xp, _P4, _K4, _E2[0], _E2[1],
                           _PI[(0, 0)], _PI[(0, 1)],
                           _PI[(F2, 0)], _PI[(F2, 1)])
    s1a, s1b = _conv1(src, dst, up(g1a), up(g1b))
    q0, q1, q2, q3 = _mid(dn(s1a), dn(s1b), dinv, bd1a, bd1b, b1t,
                          _P8, _K8, *_PIQ)
    t0, t1, t2, t3 = _conv2(src, dst, up(q0), up(q1), up(q2), up(q3))
    return _fin(dn(t0), dn(t1), dn(t2), dn(t3), dinv, bat2,
                bd2[0], bd2[1], bd2[2], bd2[3], b2t,
                _P8, _K8, Wl, bl)


# speedup vs baseline: 1.2635x; 1.2635x over previous
import functools

import numpy as np

import jax
import jax.numpy as jnp
from jax import lax
from jax.experimental import pallas as pl
from jax.experimental.pallas import tpu as pltpu
from jax.experimental.pallas import tpu_sc as plsc

N = 50000
E = 800000
VOCAB = 1000
EMB = 32
HID = 64
NCLS = 10
G = 256

NCORE = 2
NSUB = 16

NROW = 392
NROWB = 49
N_PAD = NROW * 128
DEAD = N_PAD - N
EROW = 6400
E_PAD = EROW * 128
TROW = EROW // NSUB
GRP = 8
NGRP = TROW // GRP
DROW = EROW // (NCORE * NSUB)
NODES_T = N_PAD // NSUB

_mesh = plsc.VectorSubcoreMesh(core_axis_name="c", subcore_axis_name="s",
                               num_cores=NCORE, num_subcores=NSUB)

_HIGH = lax.Precision.HIGHEST

_SC_PARAMS = pltpu.CompilerParams(use_tc_tiling_on_sc=False)


@functools.partial(
    pl.kernel,
    out_type=(jax.ShapeDtypeStruct((N_PAD, EMB), jnp.float32),
              jax.ShapeDtypeStruct((N_PAD,), jnp.float32),
              jax.ShapeDtypeStruct((N_PAD,), jnp.float32)),
    mesh=_mesh,
    compiler_params=_SC_PARAMS,
    scratch_types=[
        pltpu.VMEM((8, 128), jnp.int32),
        pltpu.VMEM((8, 128, EMB), jnp.float32),
        pltpu.VMEM((128,), jnp.float32),
        pltpu.VMEM((DROW, 128), jnp.int32),
        pltpu.VMEM((NODES_T,), jnp.float32),
        pltpu.VMEM_SHARED((N_PAD,), jnp.float32),
        pltpu.SemaphoreType.DMA,
        pltpu.SemaphoreType.DMA,
    ],
)
def _emb_deg(tok_hbm, dst_hbm, emb_hbm, x_out, deg0_out, deg1_out,
             tokb_v, erows_v, ones_v, dstb_v, zeros_v, deg_sh, sem, dsem):
    c = lax.axis_index("c")
    s = lax.axis_index("s")
    w = s * NCORE + c

    for g in range(2):
        q = g * 32 + w

        @pl.when(q < NROWB)
        def _():
            pltpu.sync_copy(tok_hbm.at[q], tokb_v)
            cps = [pltpu.async_copy(emb_hbm.at[tokb_v.at[j]], erows_v.at[j],
                                    sem) for j in range(8)]
            for cp in cps:
                cp.wait()
            for j in range(8):
                pltpu.sync_copy(erows_v.at[j],
                                x_out.at[pl.ds(q * 1024 + j * 128, 128)])

    def _z(i, _):
        zeros_v[pl.ds(i * 16, 16)] = jnp.zeros((16,), jnp.float32)
        return 0
    lax.fori_loop(0, NODES_T // 16, _z, 0)

    def _o(i, _):
        ones_v[pl.ds(i * 16, 16)] = jnp.ones((16,), jnp.float32)
        return 0
    lax.fori_loop(0, 8, _o, 0)

    pltpu.sync_copy(zeros_v, deg_sh.at[pl.ds(s * NODES_T, NODES_T)])
    plsc.subcore_barrier()

    base = w * DROW
    pltpu.sync_copy(dst_hbm.at[pl.ds(base, DROW)], dstb_v)

    def _sc(j, _):
        pltpu.async_copy(ones_v, deg_sh.at[dstb_v.at[j]], dsem, add=True)
        return 0
    lax.fori_loop(0, DROW, _sc, 0)

    def _dr(j, _):
        pltpu.make_async_copy(ones_v, deg_sh.at[dstb_v.at[j]], dsem).wait()
        return 0
    lax.fori_loop(0, DROW, _dr, 0)
    plsc.subcore_barrier()

    @pl.when(c == 0)
    def _():
        pltpu.sync_copy(deg_sh.at[pl.ds(s * NODES_T, NODES_T)],
                        deg0_out.at[pl.ds(s * NODES_T, NODES_T)])

    @pl.when(c == 1)
    def _():
        pltpu.sync_copy(deg_sh.at[pl.ds(s * NODES_T, NODES_T)],
                        deg1_out.at[pl.ds(s * NODES_T, NODES_T)])


F2 = 16


def _make_conv(f2, npass):
    grp = 5
    ngrp = TROW // grp
    assert ngrp * grp == TROW and ngrp % 4 == 0
    n_io = NCORE * npass
    NS4 = 4

    @functools.partial(
        pl.kernel,
        out_type=tuple(jax.ShapeDtypeStruct((N_PAD, f2), jnp.float32)
                       for _ in range(n_io)),
        mesh=_mesh,
        compiler_params=_SC_PARAMS,
        scratch_types=[
            pltpu.VMEM((NS4, grp, 128), jnp.int32),
            pltpu.VMEM((NS4, grp, 128), jnp.int32),
            pltpu.VMEM((NS4, grp, 128, f2), jnp.float32),
            pltpu.VMEM_SHARED((N_PAD, f2), jnp.float32),
        ] + [pltpu.SemaphoreType.DMA] * (3 * NS4),
    )
    def conv(*args):
        src_hbm, dst_hbm = args[0], args[1]
        g_hbms = args[2:2 + n_io]
        outs = args[2 + n_io:2 + 2 * n_io]
        rest = args[2 + 2 * n_io:]
        idxs_v, idxd_v, rows_v, acc_sh = rest[:4]
        sem_g = rest[4:4 + NS4]
        sem_s = rest[4 + NS4:4 + 2 * NS4]
        sem_i = rest[4 + 2 * NS4:]
        c = lax.axis_index("c")
        s = lax.axis_index("s")
        nb = s * NODES_T
        eb = s * TROW

        def run(g_hbm, out_hbm):
            pltpu.sync_copy(g_hbm.at[pl.ds(nb, NODES_T)],
                            acc_sh.at[pl.ds(nb, NODES_T)])
            plsc.subcore_barrier()

            def fire_idx(gi, b):
                ro = eb + gi * grp
                pltpu.async_copy(src_hbm.at[pl.ds(ro, grp)], idxs_v.at[b],
                                 sem_i[b])
                pltpu.async_copy(dst_hbm.at[pl.ds(ro, grp)], idxd_v.at[b],
                                 sem_i[b])

            def wait_idx(b):
                pltpu.make_async_copy(src_hbm.at[pl.ds(eb, grp)],
                                      idxs_v.at[b], sem_i[b]).wait()
                pltpu.make_async_copy(dst_hbm.at[pl.ds(eb, grp)],
                                      idxd_v.at[b], sem_i[b]).wait()

            def fire_g(b):
                for j in range(grp):
                    pltpu.async_copy(g_hbm.at[idxs_v.at[b, j]],
                                     rows_v.at[b, j], sem_g[b])

            def drain_g(b):
                for j in range(grp):
                    pltpu.make_async_copy(g_hbm.at[idxs_v.at[b, j]],
                                          rows_v.at[b, j], sem_g[b]).wait()

            def fire_s(b):
                for j in range(grp):
                    pltpu.async_copy(rows_v.at[b, j],
                                     acc_sh.at[idxd_v.at[b, j]], sem_s[b],
                                     add=True)

            def drain_s(b):
                for j in range(grp):
                    pltpu.make_async_copy(rows_v.at[b, j],
                                          acc_sh.at[idxd_v.at[b, j]],
                                          sem_s[b]).wait()

            fire_idx(0, 0)

            def quad(i, _):
                for u in range(NS4):
                    g = NS4 * i + u
                    un = (u + 1) % NS4
                    v = (u + NS4 - 1) % NS4
                    wait_idx(u)
                    fire_g(u)

                    @pl.when(g >= 1)
                    def _():
                        drain_g(v)
                        fire_s(v)

                    @pl.when(g >= 3)
                    def _():
                        drain_s(un)

                    @pl.when(g + 1 < ngrp)
                    def _():
                        fire_idx(g + 1, un)
                return 0

            lax.fori_loop(0, ngrp // NS4, quad, 0)
            last = (ngrp - 1) % NS4
            drain_g(last)
            fire_s(last)
            for gg in (ngrp - 3, ngrp - 2, ngrp - 1):
                drain_s(gg % NS4)
            plsc.subcore_barrier()
            pltpu.sync_copy(acc_sh.at[pl.ds(nb, NODES_T)],
                            out_hbm.at[pl.ds(nb, NODES_T)])

        for ci in range(NCORE):
            @pl.when(c == ci)
            def _():
                for p in range(npass):
                    run(g_hbms[ci * npass + p], outs[ci * npass + p])

    return conv


_conv1 = _make_conv(16, 1)
_conv2 = _make_conv(16, 2)


_BLK = 512
_NBLK = N_PAD // _BLK
_XB = _BLK * EMB // 128
_QB = _BLK * F2 // 128

_P4 = (np.arange(_BLK)[None, :] // 4 == np.arange(128)[:, None]
       ).astype(np.float32)
_K4 = (np.arange(128)[None, :] // 32 == (np.arange(_BLK) % 4)[:, None]
       ).astype(np.float32)
_P8 = (np.arange(_BLK)[None, :] // 8 == np.arange(64)[:, None]
       ).astype(np.float32)
_K8 = (np.arange(512)[None, :] // 64 == (np.arange(_BLK) % 8)[:, None]
       ).astype(np.float32)
_E2 = [(np.arange(128)[None, :] == 2 * np.arange(64)[:, None] + t
        ).astype(np.float32) for t in range(2)]


def _mk_pi(off, t):
    lo = np.arange(128)[None, :]
    li = np.arange(128)[:, None]
    j, f = lo // 16, lo % 16
    return ((j // 4 == t) & (li == (j % 4) * 32 + off + f)).astype(np.float32)


_PI = {(off, t): _mk_pi(off, t) for off in (0, F2) for t in range(2)}


def _mk_piq(k):
    lo = np.arange(128)[None, :]
    li = np.arange(512)[:, None]
    return (li == (lo // 16) * 64 + k * 16 + lo % 16).astype(np.float32)


_PIQ = [_mk_piq(k) for k in range(4)]


def _dot(a, b):
    return jnp.dot(a, b, precision=_HIGH, preferred_element_type=jnp.float32)


def _expand(dinv, pmat, kmat):
    return _dot(pmat * dinv[None, :], kmat)


def _prep_body(deg0_ref, deg1_ref, xp_ref, p4_ref, k4_ref, e0_ref, e1_ref,
               pa0_ref, pa1_ref, pb0_ref, pb1_ref,
               dinv_ref, g1a_ref, g1b_ref):
    deg = deg0_ref[...] + deg1_ref[...] + 1.0
    dinv = lax.rsqrt(deg)
    dinv_ref[...] = dinv
    gp = xp_ref[...] * _expand(dinv, p4_ref[...], k4_ref[...])
    e = (e0_ref[...], e1_ref[...])
    g1a_ref[...] = (_dot(_dot(e[0], gp), pa0_ref[...])
                    + _dot(_dot(e[1], gp), pa1_ref[...]))
    g1b_ref[...] = (_dot(_dot(e[0], gp), pb0_ref[...])
                    + _dot(_dot(e[1], gp), pb1_ref[...]))


_prep = pl.pallas_call(
    _prep_body,
    grid=(_NBLK,),
    in_specs=[
        pl.BlockSpec((_BLK,), lambda i: (i,)),
        pl.BlockSpec((_BLK,), lambda i: (i,)),
        pl.BlockSpec((_XB, 128), lambda i: (i, 0)),
        pl.BlockSpec((128, _BLK), lambda i: (0, 0)),
        pl.BlockSpec((_BLK, 128), lambda i: (0, 0)),
        pl.BlockSpec((_QB, 128), lambda i: (0, 0)),
        pl.BlockSpec((_QB, 128), lambda i: (0, 0)),
        pl.BlockSpec((128, 128), lambda i: (0, 0)),
        pl.BlockSpec((128, 128), lambda i: (0, 0)),
        pl.BlockSpec((128, 128), lambda i: (0, 0)),
        pl.BlockSpec((128, 128), lambda i: (0, 0)),
    ],
    out_specs=(
        pl.BlockSpec((_BLK,), lambda i: (i,)),
        pl.BlockSpec((_QB, 128), lambda i: (i, 0)),
        pl.BlockSpec((_QB, 128), lambda i: (i, 0)),
    ),
    out_shape=(jax.ShapeDtypeStruct((N_PAD,), jnp.float32),
               jax.ShapeDtypeStruct((N_PAD * F2 // 128, 128), jnp.float32),
               jax.ShapeDtypeStruct((N_PAD * F2 // 128, 128), jnp.float32)))


def _mid_body(s1a_ref, s1b_ref, dinv_ref, bd1a_ref, bd1b_ref, b1t_ref,
              p8_ref, k8_ref, piq0_ref, piq1_ref, piq2_ref, piq3_ref,
              q0_ref, q1_ref, q2_ref, q3_ref):
    dinv = dinv_ref[...]
    y = _dot(s1a_ref[...], bd1a_ref[...]) + _dot(s1b_ref[...], bd1b_ref[...])
    dinv64 = _expand(dinv, p8_ref[...], k8_ref[...])
    x1 = jnp.maximum(y * dinv64 + b1t_ref[...][None, :], 0.0)
    g2 = x1 * dinv64
    q0_ref[...] = _dot(g2, piq0_ref[...])
    q1_ref[...] = _dot(g2, piq1_ref[...])
    q2_ref[...] = _dot(g2, piq2_ref[...])
    q3_ref[...] = _dot(g2, piq3_ref[...])


_mid = pl.pallas_call(
    _mid_body,
    grid=(_NBLK,),
    in_specs=[
        pl.BlockSpec((_QB, 128), lambda i: (i, 0)),
        pl.BlockSpec((_QB, 128), lambda i: (i, 0)),
        pl.BlockSpec((_BLK,), lambda i: (i,)),
        pl.BlockSpec((128, 512), lambda i: (0, 0)),
        pl.BlockSpec((128, 512), lambda i: (0, 0)),
        pl.BlockSpec((512,), lambda i: (0,)),
        pl.BlockSpec((_QB, _BLK), lambda i: (0, 0)),
        pl.BlockSpec((_BLK, 512), lambda i: (0, 0)),
        pl.BlockSpec((512, 128), lambda i: (0, 0)),
        pl.BlockSpec((512, 128), lambda i: (0, 0)),
        pl.BlockSpec((512, 128), lambda i: (0, 0)),
        pl.BlockSpec((512, 128), lambda i: (0, 0)),
    ],
    out_specs=tuple(pl.BlockSpec((_QB, 128), lambda i: (i, 0))
                    for _ in range(4)),
    out_shape=tuple(jax.ShapeDtypeStruct((N_PAD * F2 // 128, 128),
                                         jnp.float32)
                    for _ in range(4)))


def _fin_body(q0_ref, q1_ref, q2_ref, q3_ref, dinv_ref, bat2_ref,
              bd0_ref, bd1_ref, bd2_ref, bd3_ref, b2t_ref,
              p8_ref, k8_ref, wl_ref, bl_ref, out_ref, acc_s, acc_c):
    i = pl.program_id(0)

    @pl.when(i == 0)
    def _():
        acc_s[...] = jnp.zeros_like(acc_s)
        acc_c[...] = jnp.zeros_like(acc_c)

    y = (_dot(q0_ref[...], bd0_ref[...]) + _dot(q1_ref[...], bd1_ref[...])
         + _dot(q2_ref[...], bd2_ref[...]) + _dot(q3_ref[...], bd3_ref[...]))
    dinv64 = _expand(dinv_ref[...], p8_ref[...], k8_ref[...])
    x2 = jnp.maximum(y * dinv64 + b2t_ref[...][None, :], 0.0)
    bat2 = bat2_ref[...]
    for j in range(8):
        bj = bat2[:, j]
        oh = (bj[:, None]
              == lax.broadcasted_iota(jnp.int32, (_QB, G), 1)
              ).astype(jnp.float32)
        acc_s[...] += lax.dot_general(
            oh, x2[:, j * HID:(j + 1) * HID], (((0,), (0,)), ((), ())),
            precision=_HIGH, preferred_element_type=jnp.float32)
        acc_c[...] += jnp.sum(oh, axis=0)

    @pl.when(i == pl.num_programs(0) - 1)
    def _():
        pooled = acc_s[...] / jnp.maximum(acc_c[...], 1.0)[:, None]
        out_ref[...] = (jnp.dot(pooled, wl_ref[...], precision=_HIGH,
                                preferred_element_type=jnp.float32)
                        + bl_ref[...][None, :])


_fin = pl.pallas_call(
    _fin_body,
    grid=(_NBLK,),
    in_specs=[
        pl.BlockSpec((_QB, 128), lambda i: (i, 0)),
        pl.BlockSpec((_QB, 128), lambda i: (i, 0)),
        pl.BlockSpec((_QB, 128), lambda i: (i, 0)),
        pl.BlockSpec((_QB, 128), lambda i: (i, 0)),
        pl.BlockSpec((_BLK,), lambda i: (i,)),
        pl.BlockSpec((_QB, 8), lambda i: (i, 0)),
        pl.BlockSpec((128, 512), lambda i: (0, 0)),
        pl.BlockSpec((128, 512), lambda i: (0, 0)),
        pl.BlockSpec((128, 512), lambda i: (0, 0)),
        pl.BlockSpec((128, 512), lambda i: (0, 0)),
        pl.BlockSpec((512,), lambda i: (0,)),
        pl.BlockSpec((_QB, _BLK), lambda i: (0, 0)),
        pl.BlockSpec((_BLK, 512), lambda i: (0, 0)),
        pl.BlockSpec((HID, NCLS), lambda i: (0, 0)),
        pl.BlockSpec((NCLS,), lambda i: (0,)),
    ],
    out_specs=pl.BlockSpec((G, NCLS), lambda i: (0, 0)),
    out_shape=jax.ShapeDtypeStruct((G, NCLS), jnp.float32),
    scratch_shapes=[pltpu.VMEM((G, HID), jnp.float32),
                    pltpu.VMEM((G,), jnp.float32)],
)


def kernel(x_token, edge_index, batch, embed, W1, b1, W2, b2, Wl, bl):
    npad = E_PAD - E
    dead = N + (jnp.arange(npad, dtype=jnp.int32) % DEAD)
    src = jnp.concatenate([edge_index[0], dead]).reshape(EROW, 128)
    dst = jnp.concatenate([edge_index[1], dead]).reshape(EROW, 128)
    tok = jnp.concatenate(
        [x_token, jnp.zeros((DEAD,), jnp.int32)]).reshape(NROWB, 8, 128)
    bat2 = jnp.concatenate(
        [batch, jnp.full((DEAD,), G, jnp.int32)]).reshape(N_PAD // 8, 8)

    def up(a):
        return jnp.reshape(a, (N_PAD, F2))

    def dn(a):
        return jnp.reshape(a, (N_PAD * F2 // 128, 128))

    eye8 = jnp.eye(8, dtype=jnp.float32)
    bd1a = jnp.kron(eye8, W1[:F2])
    bd1b = jnp.kron(eye8, W1[F2:])
    bd2 = [jnp.kron(eye8, W2[k * F2:(k + 1) * F2]) for k in range(4)]
    b1t = jnp.tile(b1, 8)
    b2t = jnp.tile(b2, 8)

    x, deg0, deg1 = _emb_deg(tok, dst, embed)
    xp = jnp.reshape(x, (N_PAD * EMB // 128, 128))
    dinv, g1a, g1b = _prep(deg0, deg1, xp, _P4, _K4, _E2[0], _E2[1],
                           _PI[(0, 0)], _PI[(0, 1)],
                           _PI[(F2, 0)], _PI[(F2, 1)])
    s1a, s1b = _conv1(src, dst, up(g1a), up(g1b))
    q0, q1, q2, q3 = _mid(dn(s1a), dn(s1b), dinv, bd1a, bd1b, b1t,
                          _P8, _K8, *_PIQ)
    t0, t1, t2, t3 = _conv2(src, dst, up(q0), up(q1), up(q2), up(q3))
    return _fin(dn(t0), dn(t1), dn(t2), dn(t3), dinv, bat2,
                bd2[0], bd2[1], bd2[2], bd2[3], b2t,
                _P8, _K8, Wl, bl)


# speedup vs baseline: 1.4062x; 1.1129x over previous
import functools

import numpy as np

import jax
import jax.numpy as jnp
from jax import lax
from jax.experimental import pallas as pl
from jax.experimental.pallas import tpu as pltpu
from jax.experimental.pallas import tpu_sc as plsc

N = 50000
E = 800000
VOCAB = 1000
EMB = 32
HID = 64
NCLS = 10
G = 256

NCORE = 2
NSUB = 16

NROW = 392
NROWB = 49
N_PAD = NROW * 128
DEAD = N_PAD - N
EROW = 6400
E_PAD = EROW * 128
TROW = EROW // NSUB
GRP = 8
NGRP = TROW // GRP
DROW = EROW // (NCORE * NSUB)
NODES_T = N_PAD // NSUB

_mesh = plsc.VectorSubcoreMesh(core_axis_name="c", subcore_axis_name="s",
                               num_cores=NCORE, num_subcores=NSUB)

_HIGH = lax.Precision.HIGHEST

_SC_PARAMS = pltpu.CompilerParams(use_tc_tiling_on_sc=False)


@functools.partial(
    pl.kernel,
    out_type=(jax.ShapeDtypeStruct((N_PAD, EMB), jnp.float32),
              jax.ShapeDtypeStruct((N_PAD,), jnp.float32),
              jax.ShapeDtypeStruct((N_PAD,), jnp.float32)),
    mesh=_mesh,
    compiler_params=_SC_PARAMS,
    scratch_types=[
        pltpu.VMEM((8, 128), jnp.int32),
        pltpu.VMEM((8, 128, EMB), jnp.float32),
        pltpu.VMEM((128,), jnp.float32),
        pltpu.VMEM((DROW, 128), jnp.int32),
        pltpu.VMEM((NODES_T,), jnp.float32),
        pltpu.VMEM_SHARED((N_PAD,), jnp.float32),
        pltpu.SemaphoreType.DMA,
        pltpu.SemaphoreType.DMA,
    ],
)
def _emb_deg(tok_hbm, dst_hbm, emb_hbm, x_out, deg0_out, deg1_out,
             tokb_v, erows_v, ones_v, dstb_v, zeros_v, deg_sh, sem, dsem):
    c = lax.axis_index("c")
    s = lax.axis_index("s")
    w = s * NCORE + c

    for g in range(2):
        q = g * 32 + w

        @pl.when(q < NROWB)
        def _():
            pltpu.sync_copy(tok_hbm.at[q], tokb_v)
            cps = [pltpu.async_copy(emb_hbm.at[tokb_v.at[j]], erows_v.at[j],
                                    sem) for j in range(8)]
            for cp in cps:
                cp.wait()
            for j in range(8):
                pltpu.sync_copy(erows_v.at[j],
                                x_out.at[pl.ds(q * 1024 + j * 128, 128)])

    def _z(i, _):
        zeros_v[pl.ds(i * 16, 16)] = jnp.zeros((16,), jnp.float32)
        return 0
    lax.fori_loop(0, NODES_T // 16, _z, 0)

    def _o(i, _):
        ones_v[pl.ds(i * 16, 16)] = jnp.ones((16,), jnp.float32)
        return 0
    lax.fori_loop(0, 8, _o, 0)

    pltpu.sync_copy(zeros_v, deg_sh.at[pl.ds(s * NODES_T, NODES_T)])
    plsc.subcore_barrier()

    base = w * DROW
    pltpu.sync_copy(dst_hbm.at[pl.ds(base, DROW)], dstb_v)

    def _sc(j, _):
        pltpu.async_copy(ones_v, deg_sh.at[dstb_v.at[j]], dsem, add=True)
        return 0
    lax.fori_loop(0, DROW, _sc, 0)

    def _dr(j, _):
        pltpu.make_async_copy(ones_v, deg_sh.at[dstb_v.at[j]], dsem).wait()
        return 0
    lax.fori_loop(0, DROW, _dr, 0)
    plsc.subcore_barrier()

    @pl.when(c == 0)
    def _():
        pltpu.sync_copy(deg_sh.at[pl.ds(s * NODES_T, NODES_T)],
                        deg0_out.at[pl.ds(s * NODES_T, NODES_T)])

    @pl.when(c == 1)
    def _():
        pltpu.sync_copy(deg_sh.at[pl.ds(s * NODES_T, NODES_T)],
                        deg1_out.at[pl.ds(s * NODES_T, NODES_T)])


F2 = 16


def _make_conv(f2, npass):
    grp = 5
    ngrp = TROW // grp
    assert ngrp * grp == TROW and ngrp % 4 == 0
    n_io = NCORE * npass
    NS4 = 4

    @functools.partial(
        pl.kernel,
        out_type=tuple(jax.ShapeDtypeStruct((N_PAD, f2), jnp.float32)
                       for _ in range(n_io)),
        mesh=_mesh,
        compiler_params=_SC_PARAMS,
        scratch_types=[
            pltpu.VMEM((NS4, grp, 128), jnp.int32),
            pltpu.VMEM((NS4, grp, 128), jnp.int32),
            pltpu.VMEM((NS4, grp, 128, f2), jnp.float32),
            pltpu.VMEM_SHARED((N_PAD, f2), jnp.float32),
        ] + [pltpu.SemaphoreType.DMA] * (3 * NS4),
    )
    def conv(*args):
        src_hbm, dst_hbm = args[0], args[1]
        g_hbms = args[2:2 + n_io]
        outs = args[2 + n_io:2 + 2 * n_io]
        rest = args[2 + 2 * n_io:]
        idxs_v, idxd_v, rows_v, acc_sh = rest[:4]
        sem_g = rest[4:4 + NS4]
        sem_s = rest[4 + NS4:4 + 2 * NS4]
        sem_i = rest[4 + 2 * NS4:]
        c = lax.axis_index("c")
        s = lax.axis_index("s")
        nb = s * NODES_T
        eb = s * TROW

        def run(g_hbm, out_hbm):
            pltpu.sync_copy(g_hbm.at[pl.ds(nb, NODES_T)],
                            acc_sh.at[pl.ds(nb, NODES_T)])
            plsc.subcore_barrier()

            def fire_idx(gi, b):
                ro = eb + gi * grp
                pltpu.async_copy(src_hbm.at[pl.ds(ro, grp)], idxs_v.at[b],
                                 sem_i[b])
                pltpu.async_copy(dst_hbm.at[pl.ds(ro, grp)], idxd_v.at[b],
                                 sem_i[b])

            def wait_idx(b):
                pltpu.make_async_copy(src_hbm.at[pl.ds(eb, grp)],
                                      idxs_v.at[b], sem_i[b]).wait()
                pltpu.make_async_copy(dst_hbm.at[pl.ds(eb, grp)],
                                      idxd_v.at[b], sem_i[b]).wait()

            def fire_g(b):
                for j in range(grp):
                    pltpu.async_copy(g_hbm.at[idxs_v.at[b, j]],
                                     rows_v.at[b, j], sem_g[b])

            def drain_g(b):
                for j in range(grp):
                    pltpu.make_async_copy(g_hbm.at[idxs_v.at[b, j]],
                                          rows_v.at[b, j], sem_g[b]).wait()

            def fire_s(b):
                for j in range(grp):
                    pltpu.async_copy(rows_v.at[b, j],
                                     acc_sh.at[idxd_v.at[b, j]], sem_s[b],
                                     add=True)

            def drain_s(b):
                for j in range(grp):
                    pltpu.make_async_copy(rows_v.at[b, j],
                                          acc_sh.at[idxd_v.at[b, j]],
                                          sem_s[b]).wait()

            fire_idx(0, 0)

            def quad(i, _):
                for u in range(NS4):
                    g = NS4 * i + u
                    un = (u + 1) % NS4
                    v = (u + NS4 - 1) % NS4
                    wait_idx(u)
                    fire_g(u)

                    @pl.when(g >= 1)
                    def _():
                        drain_g(v)
                        fire_s(v)

                    @pl.when(g >= 3)
                    def _():
                        drain_s(un)

                    @pl.when(g + 1 < ngrp)
                    def _():
                        fire_idx(g + 1, un)
                return 0

            lax.fori_loop(0, ngrp // NS4, quad, 0)
            last = (ngrp - 1) % NS4
            drain_g(last)
            fire_s(last)
            for gg in (ngrp - 3, ngrp - 2, ngrp - 1):
                drain_s(gg % NS4)
            plsc.subcore_barrier()
            pltpu.sync_copy(acc_sh.at[pl.ds(nb, NODES_T)],
                            out_hbm.at[pl.ds(nb, NODES_T)])

        for ci in range(NCORE):
            @pl.when(c == ci)
            def _():
                for p in range(npass):
                    run(g_hbms[ci * npass + p], outs[ci * npass + p])

    return conv


_conv1 = _make_conv(16, 1)
_conv2 = _make_conv(16, 2)


_BLK = 512
_NBLK = N_PAD // _BLK
_XB = _BLK * EMB // 128
_QB = _BLK * F2 // 128

_P4 = (np.arange(_BLK)[None, :] // 4 == np.arange(128)[:, None]
       ).astype(np.float32)
_K4 = (np.arange(128)[None, :] // 32 == (np.arange(_BLK) % 4)[:, None]
       ).astype(np.float32)
_P8 = (np.arange(_BLK)[None, :] // 8 == np.arange(64)[:, None]
       ).astype(np.float32)
_K8 = (np.arange(512)[None, :] // 64 == (np.arange(_BLK) % 8)[:, None]
       ).astype(np.float32)
_E2 = [(np.arange(128)[None, :] == 2 * np.arange(64)[:, None] + t
        ).astype(np.float32) for t in range(2)]


def _mk_pi(off, t):
    lo = np.arange(128)[None, :]
    li = np.arange(128)[:, None]
    j, f = lo // 16, lo % 16
    return ((j // 4 == t) & (li == (j % 4) * 32 + off + f)).astype(np.float32)


_PI = {(off, t): _mk_pi(off, t) for off in (0, F2) for t in range(2)}


def _mk_piq(k):
    lo = np.arange(128)[None, :]
    li = np.arange(512)[:, None]
    return (li == (lo // 16) * 64 + k * 16 + lo % 16).astype(np.float32)


_PIQ = [_mk_piq(k) for k in range(4)]

_K16 = (np.arange(128)[None, :] // 16 == (np.arange(_BLK) % 8)[:, None]
        ).astype(np.float32)
_M64 = (np.arange(128)[:, None] == (np.arange(512)[None, :] // 64) * 16
        ).astype(np.float32)


def _dot(a, b):
    return jnp.dot(a, b, precision=_HIGH, preferred_element_type=jnp.float32)


def _expand(dinv, pmat, kmat):
    return _dot(pmat * dinv[None, :], kmat)


def _prep_body(deg0_ref, deg1_ref, xp_ref, p4_ref, k4_ref, e0_ref, e1_ref,
               pa0_ref, pa1_ref, pb0_ref, pb1_ref, p8_ref, k16_ref,
               d16_ref, g1a_ref, g1b_ref):
    deg = deg0_ref[...] + deg1_ref[...] + 1.0
    dinv = lax.rsqrt(deg)
    d16_ref[...] = _expand(dinv, p8_ref[...], k16_ref[...])
    gp = xp_ref[...] * _expand(dinv, p4_ref[...], k4_ref[...])
    e = (e0_ref[...], e1_ref[...])
    g1a_ref[...] = (_dot(_dot(e[0], gp), pa0_ref[...])
                    + _dot(_dot(e[1], gp), pa1_ref[...]))
    g1b_ref[...] = (_dot(_dot(e[0], gp), pb0_ref[...])
                    + _dot(_dot(e[1], gp), pb1_ref[...]))


_prep = pl.pallas_call(
    _prep_body,
    grid=(_NBLK,),
    in_specs=[
        pl.BlockSpec((_BLK,), lambda i: (i,)),
        pl.BlockSpec((_BLK,), lambda i: (i,)),
        pl.BlockSpec((_XB, 128), lambda i: (i, 0)),
        pl.BlockSpec((128, _BLK), lambda i: (0, 0)),
        pl.BlockSpec((_BLK, 128), lambda i: (0, 0)),
        pl.BlockSpec((_QB, 128), lambda i: (0, 0)),
        pl.BlockSpec((_QB, 128), lambda i: (0, 0)),
        pl.BlockSpec((128, 128), lambda i: (0, 0)),
        pl.BlockSpec((128, 128), lambda i: (0, 0)),
        pl.BlockSpec((128, 128), lambda i: (0, 0)),
        pl.BlockSpec((128, 128), lambda i: (0, 0)),
        pl.BlockSpec((_QB, _BLK), lambda i: (0, 0)),
        pl.BlockSpec((_BLK, 128), lambda i: (0, 0)),
    ],
    out_specs=(
        pl.BlockSpec((_QB, 128), lambda i: (i, 0)),
        pl.BlockSpec((_QB, 128), lambda i: (i, 0)),
        pl.BlockSpec((_QB, 128), lambda i: (i, 0)),
    ),
    out_shape=(jax.ShapeDtypeStruct((N_PAD * F2 // 128, 128), jnp.float32),
               jax.ShapeDtypeStruct((N_PAD * F2 // 128, 128), jnp.float32),
               jax.ShapeDtypeStruct((N_PAD * F2 // 128, 128), jnp.float32)))


def _mid_body(s1a_ref, s1b_ref, d16_ref, bd1a_ref, bd1b_ref, b1t_ref,
              piq0_ref, piq1_ref, piq2_ref, piq3_ref,
              q0_ref, q1_ref, q2_ref, q3_ref):
    d16 = d16_ref[...]
    y = (_dot(s1a_ref[...] * d16, bd1a_ref[...])
         + _dot(s1b_ref[...] * d16, bd1b_ref[...]))
    x1 = jnp.maximum(y + b1t_ref[...][None, :], 0.0)
    q0_ref[...] = _dot(x1, piq0_ref[...]) * d16
    q1_ref[...] = _dot(x1, piq1_ref[...]) * d16
    q2_ref[...] = _dot(x1, piq2_ref[...]) * d16
    q3_ref[...] = _dot(x1, piq3_ref[...]) * d16


_mid = pl.pallas_call(
    _mid_body,
    grid=(_NBLK,),
    in_specs=[
        pl.BlockSpec((_QB, 128), lambda i: (i, 0)),
        pl.BlockSpec((_QB, 128), lambda i: (i, 0)),
        pl.BlockSpec((_QB, 128), lambda i: (i, 0)),
        pl.BlockSpec((128, 512), lambda i: (0, 0)),
        pl.BlockSpec((128, 512), lambda i: (0, 0)),
        pl.BlockSpec((512,), lambda i: (0,)),
        pl.BlockSpec((512, 128), lambda i: (0, 0)),
        pl.BlockSpec((512, 128), lambda i: (0, 0)),
        pl.BlockSpec((512, 128), lambda i: (0, 0)),
        pl.BlockSpec((512, 128), lambda i: (0, 0)),
    ],
    out_specs=tuple(pl.BlockSpec((_QB, 128), lambda i: (i, 0))
                    for _ in range(4)),
    out_shape=tuple(jax.ShapeDtypeStruct((N_PAD * F2 // 128, 128),
                                         jnp.float32)
                    for _ in range(4)))


def _fin_body(q0_ref, q1_ref, q2_ref, q3_ref, d16_ref, bat2_ref,
              bd0_ref, bd1_ref, bd2_ref, bd3_ref, b2t_ref,
              m64_ref, wl_ref, bl_ref, out_ref, acc_s, acc_c):
    i = pl.program_id(0)

    @pl.when(i == 0)
    def _():
        acc_s[...] = jnp.zeros_like(acc_s)
        acc_c[...] = jnp.zeros_like(acc_c)

    d16 = d16_ref[...]
    y = (_dot(q0_ref[...] * d16, bd0_ref[...])
         + _dot(q1_ref[...] * d16, bd1_ref[...])
         + _dot(q2_ref[...] * d16, bd2_ref[...])
         + _dot(q3_ref[...] * d16, bd3_ref[...]))
    dinv64 = _dot(d16, m64_ref[...])
    x2 = jnp.maximum(y * dinv64 + b2t_ref[...][None, :], 0.0)
    bat2 = bat2_ref[...]
    for j in range(8):
        bj = bat2[:, j]
        oh = (bj[:, None]
              == lax.broadcasted_iota(jnp.int32, (_QB, G), 1)
              ).astype(jnp.float32)
        acc_s[...] += lax.dot_general(
            oh, x2[:, j * HID:(j + 1) * HID], (((0,), (0,)), ((), ())),
            precision=_HIGH, preferred_element_type=jnp.float32)
        acc_c[...] += jnp.sum(oh, axis=0)

    @pl.when(i == pl.num_programs(0) - 1)
    def _():
        pooled = acc_s[...] / jnp.maximum(acc_c[...], 1.0)[:, None]
        out_ref[...] = (jnp.dot(pooled, wl_ref[...], precision=_HIGH,
                                preferred_element_type=jnp.float32)
                        + bl_ref[...][None, :])


_fin = pl.pallas_call(
    _fin_body,
    grid=(_NBLK,),
    in_specs=[
        pl.BlockSpec((_QB, 128), lambda i: (i, 0)),
        pl.BlockSpec((_QB, 128), lambda i: (i, 0)),
        pl.BlockSpec((_QB, 128), lambda i: (i, 0)),
        pl.BlockSpec((_QB, 128), lambda i: (i, 0)),
        pl.BlockSpec((_QB, 128), lambda i: (i, 0)),
        pl.BlockSpec((_QB, 8), lambda i: (i, 0)),
        pl.BlockSpec((128, 512), lambda i: (0, 0)),
        pl.BlockSpec((128, 512), lambda i: (0, 0)),
        pl.BlockSpec((128, 512), lambda i: (0, 0)),
        pl.BlockSpec((128, 512), lambda i: (0, 0)),
        pl.BlockSpec((512,), lambda i: (0,)),
        pl.BlockSpec((128, 512), lambda i: (0, 0)),
        pl.BlockSpec((HID, NCLS), lambda i: (0, 0)),
        pl.BlockSpec((NCLS,), lambda i: (0,)),
    ],
    out_specs=pl.BlockSpec((G, NCLS), lambda i: (0, 0)),
    out_shape=jax.ShapeDtypeStruct((G, NCLS), jnp.float32),
    scratch_shapes=[pltpu.VMEM((G, HID), jnp.float32),
                    pltpu.VMEM((G,), jnp.float32)],
)


def kernel(x_token, edge_index, batch, embed, W1, b1, W2, b2, Wl, bl):
    npad = E_PAD - E
    dead = N + (jnp.arange(npad, dtype=jnp.int32) % DEAD)
    src = jnp.concatenate([edge_index[0], dead]).reshape(EROW, 128)
    dst = jnp.concatenate([edge_index[1], dead]).reshape(EROW, 128)
    tok = jnp.concatenate(
        [x_token, jnp.zeros((DEAD,), jnp.int32)]).reshape(NROWB, 8, 128)
    bat2 = jnp.concatenate(
        [batch, jnp.full((DEAD,), G, jnp.int32)]).reshape(N_PAD // 8, 8)

    def up(a):
        return jnp.reshape(a, (N_PAD, F2))

    def dn(a):
        return jnp.reshape(a, (N_PAD * F2 // 128, 128))

    eye8 = jnp.eye(8, dtype=jnp.float32)
    bd1a = jnp.kron(eye8, W1[:F2])
    bd1b = jnp.kron(eye8, W1[F2:])
    bd2 = [jnp.kron(eye8, W2[k * F2:(k + 1) * F2]) for k in range(4)]
    b1t = jnp.tile(b1, 8)
    b2t = jnp.tile(b2, 8)

    x, deg0, deg1 = _emb_deg(tok, dst, embed)
    xp = jnp.reshape(x, (N_PAD * EMB // 128, 128))
    d16, g1a, g1b = _prep(deg0, deg1, xp, _P4, _K4, _E2[0], _E2[1],
                          _PI[(0, 0)], _PI[(0, 1)],
                          _PI[(F2, 0)], _PI[(F2, 1)], _P8, _K16)
    s1a, s1b = _conv1(src, dst, up(g1a), up(g1b))
    q0, q1, q2, q3 = _mid(dn(s1a), dn(s1b), d16, bd1a, bd1b, b1t, *_PIQ)
    t0, t1, t2, t3 = _conv2(src, dst, up(q0), up(q1), up(q2), up(q3))
    return _fin(dn(t0), dn(t1), dn(t2), dn(t3), d16, bat2,
                bd2[0], bd2[1], bd2[2], bd2[3], b2t,
                _M64, Wl, bl)
